# Initial kernel scaffold; baseline (speedup 1.0000x reference)
import jax
import jax.numpy as jnp
from jax.experimental import pallas as pl


def kernel(x, emb_table, conv_W1, conv_b1, conv_W2, conv_b2, pool_W1, pool_b1, pool_W2, pool_b2, edge_W1, edge_b1, edge_W2, edge_b2):
    raise NotImplementedError("write your pallas kernel here")


# speedup vs baseline: 1.4703x; 1.4703x over previous
import functools

import jax
import jax.numpy as jnp
from jax.experimental import pallas as pl
from jax.experimental.pallas import tpu as pltpu

_B, _N = 4, 256
_N_SPECIES = 10
_EMB = 128
_N_RBF = 16
_RC = 5.0
_K = 16
_N_CONV = 3
_UPDATE_RATE = 0.5
_DECAY = 0.9
_H = 64
_P_OUT = 32
_E_OUT = 32
_CH = 32
_NCH = _N // _CH
_BIG = 1e30
_TAKEN = 3e30


def _silu(v):
    return v * jax.nn.sigmoid(v)


def _tdnnx_kernel(x_ref, emb_ref, cw1_ref, cb1_ref, cw2_ref, cb2_ref,
                  pw1_ref, pb1_ref, pw2_ref, pb2_ref,
                  ew1_ref, eb1_ref, ew2_ref, eb2_ref,
                  ciso_ref, cani_ref,
                  filt_ref, d_ref, msg_ref):
    f32 = jnp.float32
    xb = x_ref[0]
    pos = xb[:, 1:4]
    spi = jnp.clip(xb[:, 0:1].astype(jnp.int32), 0, _N_SPECIES - 1)

    sp_iota = jax.lax.broadcasted_iota(jnp.int32, (_N, _N_SPECIES), 1)
    oh_sp = (spi == sp_iota).astype(f32)
    feats0 = jax.lax.dot_general(
        oh_sp, emb_ref[:], (((1,), (0,)), ((), ())),
        precision=jax.lax.Precision.HIGHEST)

    c_iota = jax.lax.broadcasted_iota(jnp.int32, (1, 1, _N_RBF), 2)
    centers3 = 0.5 + c_iota.astype(f32) * ((_RC - 0.5) / (_N_RBF - 1))

    def build_chunk(c, _):
        base = c * _CH
        xc = x_ref[0, pl.ds(base, _CH), :]
        pi = xc[:, 1:4]
        diff = pi[:, None, :] - pos[None, :, :]
        dx = diff[..., 0]
        dy = diff[..., 1]
        dz = diff[..., 2]
        d2 = dx * dx + dy * dy + dz * dz
        d = jnp.sqrt(d2 + 1e-12)
        d_ref[pl.ds(base, _CH), :] = d
        rows = base + jax.lax.broadcasted_iota(jnp.int32, (_CH, _N), 0)
        cols = jax.lax.broadcasted_iota(jnp.int32, (_CH, _N), 1)
        mask = (d < _RC) & (rows != cols)
        arg = d[..., None] - centers3
        rbf = jnp.exp(arg * arg * (-1.0 / 0.5))
        h = _silu(
            jnp.dot(rbf.reshape(_CH * _N, _N_RBF), cw1_ref[:],
                    preferred_element_type=f32) + cb1_ref[:])
        filt = (jnp.dot(h, cw2_ref[:], preferred_element_type=f32)
                + cb2_ref[:]).reshape(_CH, _N, _EMB)
        filt_ref[pl.ds(base, _CH)] = filt * mask[..., None].astype(f32)
        return 0

    jax.lax.fori_loop(0, _NCH, build_chunk, 0)

    feats = feats0
    for t in range(_N_CONV):
        def msg_chunk(c, _, feats=feats):
            base = c * _CH
            fc = filt_ref[pl.ds(base, _CH)]
            msg_ref[pl.ds(base, _CH), :] = jnp.sum(fc * feats[None, :, :],
                                                   axis=1)
            return 0
        jax.lax.fori_loop(0, _NCH, msg_chunk, 0)
        feats = feats + (_UPDATE_RATE * (_DECAY ** t)) * msg_ref[:]

    pn = jnp.dot(_silu(jnp.dot(feats, pw1_ref[:], preferred_element_type=f32)
                       + pb1_ref[:]),
                 pw2_ref[:], preferred_element_type=f32) + pb2_ref[:]
    ciso_ref[0, 0, :] = jnp.sum(pn, axis=0)

    d = d_ref[:]
    rows = jax.lax.broadcasted_iota(jnp.int32, (_N, _N), 0)
    cols = jax.lax.broadcasted_iota(jnp.int32, (_N, _N), 1)
    dm = jnp.where((d < _RC) & (rows != cols), d, _BIG)
    topi_cols = []
    for t in range(_K):
        curmin = jnp.min(dm, axis=1)
        hit = dm == curmin[:, None]
        idx = jnp.min(jnp.where(hit, cols, _N), axis=1)
        topi_cols.append(idx)
        dm = jnp.where(cols == idx[:, None], _TAKEN, dm)

    outs = []
    for k in range(_K):
        oh_k = (topi_cols[k][:, None] == cols).astype(f32)
        nbf_k = jax.lax.dot_general(oh_k, feats, (((1,), (0,)), ((), ())),
                                    precision=jax.lax.Precision.HIGHEST)
        nbp_k = jax.lax.dot_general(oh_k, pos, (((1,), (0,)), ((), ())),
                                    precision=jax.lax.Precision.HIGHEST)
        vecs = pos - nbp_k
        vx = vecs[:, 0]
        vy = vecs[:, 1]
        vz = vecs[:, 2]
        dd = jnp.sqrt(vx * vx + vy * vy + vz * vz + 1e-12)
        u = vecs / (dd[:, None] + 1e-9)
        ef = jnp.dot(
            _silu(jnp.dot(nbf_k, ew1_ref[:], preferred_element_type=f32)
                  + eb1_ref[:]),
            ew2_ref[:], preferred_element_type=f32) + eb2_ref[:]
        outs.append(u[:, :, None] * ef[:, None, :])
    cani_ref[0] = jnp.concatenate(outs, axis=1)


@jax.jit
def kernel(x, emb_table, conv_W1, conv_b1, conv_W2, conv_b2,
           pool_W1, pool_b1, pool_W2, pool_b2,
           edge_W1, edge_b1, edge_W2, edge_b2):
    f32 = jnp.float32
    full = lambda shape: pl.BlockSpec(shape, lambda b: (0,) * len(shape))
    in_specs = [
        pl.BlockSpec((1, _N, 4), lambda b: (b, 0, 0)),
        full((_N_SPECIES, _EMB)),
        full((_N_RBF, _H)), full((_H,)),
        full((_H, _EMB)), full((_EMB,)),
        full((_EMB, _H)), full((_H,)),
        full((_H, _P_OUT)), full((_P_OUT,)),
        full((_EMB, _H)), full((_H,)),
        full((_H, _E_OUT)), full((_E_OUT,)),
    ]
    out_specs = [
        pl.BlockSpec((1, 1, _P_OUT), lambda b: (b, 0, 0)),
        pl.BlockSpec((1, _N, _K * 3, _E_OUT), lambda b: (b, 0, 0, 0)),
    ]
    c_iso, c_aniso = pl.pallas_call(
        _tdnnx_kernel,
        grid=(_B,),
        in_specs=in_specs,
        out_specs=out_specs,
        out_shape=[
            jax.ShapeDtypeStruct((_B, 1, _P_OUT), f32),
            jax.ShapeDtypeStruct((_B, _N, _K * 3, _E_OUT), f32),
        ],
        scratch_shapes=[
            pltpu.VMEM((_N, _N, _EMB), f32),
            pltpu.VMEM((_N, _N), f32),
            pltpu.VMEM((_N, _EMB), f32),
        ],
        compiler_params=pltpu.CompilerParams(
            dimension_semantics=("arbitrary",),
            vmem_limit_bytes=100 * 1024 * 1024,
        ),
    )(x, emb_table, conv_W1, conv_b1, conv_W2, conv_b2,
      pool_W1, pool_b1, pool_W2, pool_b2,
      edge_W1, edge_b1, edge_W2, edge_b2)
    return c_iso.reshape(_B, _P_OUT), c_aniso
